# baseline (speedup 1.0000x reference)
import functools

import jax
import jax.numpy as jnp
from jax import lax
from jax.experimental import pallas as pl
from jax.experimental.pallas import tpu as pltpu
from jax.experimental.pallas import tpu_sc as plsc

N = 8388608
NC, NS, L = 2, 16, 16
NW = NC * NS
C = N // NW
CL = C // L
S = 1024
WIN = CL // S
WE = L * S
RB = WE // 128
SROW = S // 128
BINS = 256
SIGN = -2147483648

_mesh = plsc.VectorSubcoreMesh(
    core_axis_name="c", subcore_axis_name="s", num_cores=NC, num_subcores=NS)
_params = pltpu.CompilerParams(
    needs_layout_passes=False, use_tc_tiling_on_sc=False)


def _wid():
    return lax.axis_index("c") * NS + lax.axis_index("s")


def _digit(key, shift):
    b = key ^ SIGN
    if shift:
        b = lax.shift_right_logical(b, jnp.full((L,), shift, jnp.int32))
    return b & 255


def _fire_loads(src, w, win, dst, sem):
    for l in range(L):
        pltpu.make_async_copy(
            src.at[w * L + l, win], dst.at[pl.ds(l * SROW, SROW)], sem).start()


def _drain_loads(src, w, win, dst, sem):
    for l in range(L):
        pltpu.make_async_copy(
            src.at[w * L + l, win], dst.at[pl.ds(l * SROW, SROW)], sem).wait()


def _hist_body(shift, keys_h, hw_out, hl_out, kbuf, hl, hwrow, sem):
    w = _wid()
    iota = lax.iota(jnp.int32, L)
    zeros = jnp.zeros((L,), jnp.int32)
    ones = jnp.ones((L,), jnp.int32)
    iotaR = iota * SROW
    hbase = iota * BINS

    def zb(i, _):
        hl[pl.ds(i * L, L)] = zeros
        return 0
    lax.fori_loop(0, BINS * L // L, zb, 0)

    def win_body(win, _):
        _fire_loads(keys_h, w, win, kbuf, sem)
        _drain_loads(keys_h, w, win, kbuf, sem)

        def step(jj, _):
            row = iotaR + lax.shift_right_logical(jj, 7)
            col = jnp.broadcast_to(jj & 127, (L,))
            key = plsc.load_gather(kbuf, [row, col])
            d = _digit(key, shift)
            plsc.addupdate_scatter(hl, [hbase + d], ones)
            return 0
        lax.fori_loop(0, S, step, 0)
        return 0
    lax.fori_loop(0, WIN, win_body, 0)

    def fold(dg, _):
        def lanes(l, acc):
            return acc + hl[pl.ds(l * BINS + dg * L, L)]
        hwrow[pl.ds(dg * L, L)] = lax.fori_loop(0, L, lanes, zeros)
        return 0
    lax.fori_loop(0, BINS // L, fold, 0)

    pltpu.sync_copy(hwrow, hw_out.at[w])
    pltpu.sync_copy(hl, hl_out.at[w])


def _make_hist(shift):
    return pl.kernel(
        functools.partial(_hist_body, shift),
        out_type=(jax.ShapeDtypeStruct((NW, BINS), jnp.int32),
                  jax.ShapeDtypeStruct((NW, L * BINS), jnp.int32)),
        mesh=_mesh,
        compiler_params=_params,
        scratch_types=[
            pltpu.VMEM((RB, 128), jnp.int32),
            pltpu.VMEM((L * BINS,), jnp.int32),
            pltpu.VMEM((BINS,), jnp.int32),
            pltpu.SemaphoreType.DMA,
        ],
    )


def _offsets(w, hw, hlw, offs):
    iota = lax.iota(jnp.int32, L)
    zeros = jnp.zeros((L,), jnp.int32)
    wvec = jnp.broadcast_to(w, (L,))

    def dg_body(dg, running):
        def acc_w(ww, carry):
            tot, pref = carry
            rowv = hw[pl.ds(ww * BINS + dg * L, L)]
            mask = jnp.broadcast_to(ww, (L,)) < wvec
            return tot + rowv, pref + jnp.where(mask, rowv, zeros)
        tot, pref = lax.fori_loop(0, NW, acc_w, (zeros, zeros))
        inc = plsc.cumsum(tot)
        base = (inc - tot) + jnp.broadcast_to(running, (L,)) + pref

        def lane_body(l, acc):
            offs[pl.ds(l * BINS + dg * L, L)] = acc
            return acc + hlw[pl.ds(l * BINS + dg * L, L)]
        lax.fori_loop(0, L, lane_body, base)
        return running + jnp.sum(tot)
    lax.fori_loop(0, BINS // L, dg_body, jnp.int32(0))


def _perm_body(shift, keys_h, vals_h, hw_h, hl_h, kout, vout,
               kbuf, vbuf, dbuf, offs, hw, hlw, sem_l, sem_s):
    w = _wid()
    iota = lax.iota(jnp.int32, L)
    ones = jnp.ones((L,), jnp.int32)
    iotaR = iota * SROW
    obase = iota * BINS

    pltpu.sync_copy(hw_h, hw)
    pltpu.sync_copy(hl_h.at[w], hlw)
    _offsets(w, hw, hlw, offs)

    def win_body(win, _):
        _fire_loads(keys_h, w, win, kbuf, sem_l)
        _fire_loads(vals_h, w, win, vbuf, sem_l)
        _drain_loads(keys_h, w, win, kbuf, sem_l)
        _drain_loads(vals_h, w, win, vbuf, sem_l)

        def step(jj, _):
            row = iotaR + lax.shift_right_logical(jj, 7)
            col = jnp.broadcast_to(jj & 127, (L,))
            key = plsc.load_gather(kbuf, [row, col])
            d = _digit(key, shift)
            off = plsc.load_gather(offs, [obase + d])
            plsc.store_scatter(offs, [obase + d], off + ones)
            plsc.store_scatter(dbuf, [row, col], off)
            return 0
        lax.fori_loop(0, S, step, 0)

        def grp(g, _):
            for i in range(4):
                ch = g * 4 + i
                pltpu.make_async_copy(kbuf.at[ch], kout.at[dbuf.at[ch]], sem_s).start()
                pltpu.make_async_copy(vbuf.at[ch], vout.at[dbuf.at[ch]], sem_s).start()
            for i in range(4):
                ch = g * 4 + i
                pltpu.make_async_copy(kbuf.at[ch], kout.at[dbuf.at[ch]], sem_s).wait()
                pltpu.make_async_copy(vbuf.at[ch], vout.at[dbuf.at[ch]], sem_s).wait()
            return 0
        lax.fori_loop(0, RB // 4, grp, 0)
        return 0
    lax.fori_loop(0, WIN, win_body, 0)


def _make_perm(shift):
    return pl.kernel(
        functools.partial(_perm_body, shift),
        out_type=(jax.ShapeDtypeStruct((N,), jnp.int32),
                  jax.ShapeDtypeStruct((N,), jnp.int32)),
        mesh=_mesh,
        compiler_params=_params,
        scratch_types=[
            pltpu.VMEM((RB, 128), jnp.int32),
            pltpu.VMEM((RB, 128), jnp.int32),
            pltpu.VMEM((RB, 128), jnp.int32),
            pltpu.VMEM((L * BINS,), jnp.int32),
            pltpu.VMEM((NW * BINS,), jnp.int32),
            pltpu.VMEM((L * BINS,), jnp.int32),
            pltpu.SemaphoreType.DMA,
            pltpu.SemaphoreType.DMA,
        ],
    )


_hists = [_make_hist(8 * p) for p in range(4)]
_perms = [_make_perm(8 * p) for p in range(4)]


def kernel(keys, values):
    k, v = keys, values
    for p in range(4):
        kview = k.reshape(NW * L, WIN, SROW, 128)
        vview = v.reshape(NW * L, WIN, SROW, 128)
        hw, hl = _hists[p](kview)
        k, v = _perms[p](kview, vview, hw.reshape(-1), hl)
    return (k, v)
